# baseline (speedup 1.0000x reference)
import functools

import jax
import jax.numpy as jnp
from jax import lax
from jax.experimental import pallas as pl
from jax.experimental.pallas import tpu as pltpu
from jax.experimental.pallas import tpu_sc as plsc

_NUM_CORES = 2
_NUM_SUBCORES = 16
_NUM_WORKERS = _NUM_CORES * _NUM_SUBCORES
_CHUNK_B = 2
_NBUF = 2


def _split_s(s: int):
    spans = []
    off = 0
    while s - off > 128:
        spans.append((off, 128))
        off += 128
    rem = s - off
    if rem % 8:
        steal = 8 - rem % 8
        o, w = spans[-1]
        spans[-1] = (o, w - steal)
        off -= steal
        rem += steal
    spans.append((off, rem))
    return spans


@functools.lru_cache(maxsize=None)
def _make_lookup(bsz: int, s: int, dim: int):
    b_per_w = bsz // _NUM_WORKERS
    n_chunks = b_per_w // _CHUNK_B
    n_groups = n_chunks // _NBUF
    assert b_per_w == n_chunks * _CHUNK_B and n_chunks == n_groups * _NBUF
    spans = _split_s(s)
    mesh = plsc.VectorSubcoreMesh(core_axis_name="c", subcore_axis_name="s")

    @functools.partial(
        pl.kernel,
        out_type=jax.ShapeDtypeStruct((bsz, s, dim), jnp.float32),
        mesh=mesh,
        scratch_types=[
            pltpu.VMEM((_NBUF, _CHUNK_B, s), jnp.int32),
            pltpu.VMEM((_NBUF, _CHUNK_B, s, dim), jnp.float32),
        ]
        + [pltpu.SemaphoreType.DMA] * (3 * _NBUF),
        compiler_params=pltpu.CompilerParams(use_tc_tiling_on_sc=False),
    )
    def lookup(idx_hbm, table_hbm, out_hbm, idx_v, rows_v, *sems):
        isem = sems[:_NBUF]
        gsem = sems[_NBUF:2 * _NBUF]
        osem = sems[2 * _NBUF:]
        wid = lax.axis_index("s") * _NUM_CORES + lax.axis_index("c")
        b_base = wid * b_per_w

        def fetch_idx(c, nb):
            return pltpu.async_copy(
                idx_hbm.at[pl.ds(b_base + c * _CHUNK_B, _CHUNK_B)],
                idx_v.at[nb], isem[nb])

        def run_gathers(nb):
            copies = [
                pltpu.async_copy(
                    table_hbm.at[idx_v.at[nb, r, pl.ds(off, w)]],
                    rows_v.at[nb, r, pl.ds(off, w)],
                    gsem[nb],
                )
                for r in range(_CHUNK_B)
                for off, w in spans
            ]
            for cp in copies:
                cp.wait()

        def put_out(c, nb):
            return pltpu.async_copy(
                rows_v.at[nb],
                out_hbm.at[pl.ds(b_base + c * _CHUNK_B, _CHUNK_B)],
                osem[nb])

        def drain_out(nb):
            pltpu.make_async_copy(
                rows_v.at[nb],
                out_hbm.at[pl.ds(b_base, _CHUNK_B)],
                osem[nb]).wait()

        def drain_idx(nb):
            pltpu.make_async_copy(
                idx_hbm.at[pl.ds(b_base, _CHUNK_B)],
                idx_v.at[nb], isem[nb]).wait()

        for nb in range(_NBUF):
            fetch_idx(nb, nb)
        for nb in range(_NBUF):
            drain_idx(nb)
            run_gathers(nb)
            fetch_idx(nb + _NBUF, nb)
            put_out(nb, nb)

        def group(g, carry):
            for nb in range(_NBUF):
                c = g * _NBUF + nb
                drain_idx(nb)
                drain_out(nb)
                run_gathers(nb)
                fetch_idx(lax.min(c + _NBUF, n_chunks - 1), nb)
                put_out(c, nb)
            return carry

        lax.fori_loop(1, n_groups, group, None)

        for nb in range(_NBUF):
            drain_idx(nb)
            drain_out(nb)

    return lookup


def kernel(token_ids, weight):
    b, s = token_ids.shape
    dim = weight.shape[1]
    return _make_lookup(b, s, dim)(token_ids.astype(jnp.int32), weight)
